# Initial kernel scaffold; baseline (speedup 1.0000x reference)
import jax
import jax.numpy as jnp
from jax.experimental import pallas as pl


def kernel(inputs, normalized_adj, weights_0, bias_0):
    raise NotImplementedError("write your pallas kernel here")


# speedup vs baseline: 3.5559x; 3.5559x over previous
import jax
import jax.numpy as jnp
from jax.experimental import pallas as pl


def _gcn_body(x_ref, a_ref, w_ref, b_ref, o_ref):
    xb = x_ref[0].astype(jnp.bfloat16)
    tmp = jnp.dot(a_ref[...], xb, preferred_element_type=jnp.float32)
    out = jnp.dot(tmp.astype(jnp.bfloat16), w_ref[...],
                  preferred_element_type=jnp.float32)
    o_ref[0] = out + b_ref[...]


def kernel(inputs, normalized_adj, weights_0, bias_0):
    b, t, n, d = inputs.shape
    hid = weights_0.shape[1]
    x = inputs.reshape(b * t, n, d)
    a_bf = normalized_adj.astype(jnp.bfloat16)
    w_bf = weights_0.astype(jnp.bfloat16)
    bias2 = bias_0.reshape(1, hid)

    out = pl.pallas_call(
        _gcn_body,
        grid=(b * t,),
        in_specs=[
            pl.BlockSpec((1, n, d), lambda i: (i, 0, 0)),
            pl.BlockSpec((n, n), lambda i: (0, 0)),
            pl.BlockSpec((d, hid), lambda i: (0, 0)),
            pl.BlockSpec((1, hid), lambda i: (0, 0)),
        ],
        out_specs=pl.BlockSpec((1, n, hid), lambda i: (i, 0, 0)),
        out_shape=jax.ShapeDtypeStruct((b * t, n, hid), jnp.float32),
    )(x, a_bf, w_bf, bias2)
    return out.reshape(b, t, n, hid)
